# Initial kernel scaffold; baseline (speedup 1.0000x reference)
import jax
import jax.numpy as jnp
from jax.experimental import pallas as pl


def kernel(features, edge_index, W1, b1, W2, b2):
    raise NotImplementedError("write your pallas kernel here")


# speedup vs baseline: 9.3114x; 9.3114x over previous
import functools

import jax
import jax.numpy as jnp
from jax import lax
from jax.experimental import pallas as pl
from jax.experimental.pallas import tpu as pltpu
from jax.experimental.pallas import tpu_sc as plsc

N = 10000
E = 320000
D_IN = 128
D_HID = 128
N_CLASS = 64

NC = 2
NS = 16
NW = NC * NS
CHUNK = 128
NCHUNKS = -(-E // (NW * CHUNK))
E_PAD = NW * NCHUNKS * CHUNK
PAD_NODE = N

N_PAD = 10112
ROWS_PER_TILE = N_PAD // NS
TAIL = ROWS_PER_TILE - 4 * CHUNK
BR = 128

_MESH = plsc.VectorSubcoreMesh(core_axis_name="c", subcore_axis_name="s")


def _make_agg(d):
    @functools.partial(
        pl.kernel,
        out_type=jax.ShapeDtypeStruct((NC, N_PAD, d), jnp.float32),
        mesh=_MESH,
        scratch_types=[
            pltpu.VMEM((2, CHUNK), jnp.int32),
            pltpu.VMEM((2, CHUNK), jnp.int32),
            pltpu.VMEM((2, CHUNK, d), jnp.float32),
            pltpu.VMEM((CHUNK, d), jnp.float32),
            pltpu.VMEM_SHARED((N_PAD, d), jnp.float32),
        ],
    )
    def agg(xw_hbm, src_hbm, dst_hbm, zeros_hbm, out_hbm,
            src_v, dst_v, rows_v, zero_v, acc_sh):
        cid = lax.axis_index("c")
        sid = lax.axis_index("s")
        wid = sid * NC + cid

        pltpu.sync_copy(zeros_hbm, zero_v)

        @pl.loop(0, 4 * CHUNK, step=CHUNK)
        def _(r):
            pltpu.sync_copy(zero_v, acc_sh.at[pl.ds(sid * ROWS_PER_TILE + r, CHUNK)])

        pltpu.sync_copy(zero_v.at[pl.ds(0, TAIL)],
                        acc_sh.at[pl.ds(sid * ROWS_PER_TILE + 4 * CHUNK, TAIL)])

        plsc.subcore_barrier()

        @pl.loop(0, NCHUNKS)
        def _(j):
            pltpu.sync_copy(src_hbm.at[wid, j], src_v.at[0])
            pltpu.sync_copy(dst_hbm.at[wid, j], dst_v.at[0])
            pltpu.sync_copy(xw_hbm.at[src_v.at[0]], rows_v.at[0])
            pltpu.sync_copy(rows_v.at[0], acc_sh.at[dst_v.at[0]], add=True)

        plsc.subcore_barrier()

        @pl.loop(0, 4 * CHUNK, step=CHUNK)
        def _(r):
            base = sid * ROWS_PER_TILE + r
            pltpu.sync_copy(acc_sh.at[pl.ds(base, CHUNK)], rows_v.at[0])
            pltpu.sync_copy(rows_v.at[0], out_hbm.at[cid, pl.ds(base, CHUNK)])

        tbase = sid * ROWS_PER_TILE + 4 * CHUNK
        pltpu.sync_copy(acc_sh.at[pl.ds(tbase, TAIL)],
                        rows_v.at[0, pl.ds(0, TAIL)])
        pltpu.sync_copy(rows_v.at[0, pl.ds(0, TAIL)],
                        out_hbm.at[cid, pl.ds(tbase, TAIL)])

    return agg


_agg_hid = _make_agg(D_HID)


@functools.partial(
    pl.kernel,
    out_type=jax.ShapeDtypeStruct((NC * N_PAD,), jnp.float32),
    mesh=_MESH,
    scratch_types=[
        pltpu.VMEM((2, CHUNK), jnp.int32),
        pltpu.VMEM((2, CHUNK), jnp.float32),
        pltpu.VMEM((CHUNK,), jnp.float32),
        pltpu.VMEM_SHARED((N_PAD,), jnp.float32),
    ],
)
def _deg_kernel(dst_hbm, zo_hbm, out_hbm, idx_v, zo_v, stage_v, acc_sh):
    cid = lax.axis_index("c")
    sid = lax.axis_index("s")
    wid = sid * NC + cid

    pltpu.sync_copy(zo_hbm, zo_v)

    @pl.loop(0, 4 * CHUNK, step=CHUNK)
    def _(r):
        pltpu.sync_copy(zo_v.at[0], acc_sh.at[pl.ds(sid * ROWS_PER_TILE + r, CHUNK)])

    pltpu.sync_copy(zo_v.at[0, pl.ds(0, TAIL)],
                    acc_sh.at[pl.ds(sid * ROWS_PER_TILE + 4 * CHUNK, TAIL)])

    plsc.subcore_barrier()

    @pl.loop(0, NCHUNKS)
    def _(j):
        pltpu.sync_copy(dst_hbm.at[wid, j], idx_v.at[0])
        pltpu.sync_copy(zo_v.at[1], acc_sh.at[idx_v.at[0]], add=True)

    plsc.subcore_barrier()

    @pl.loop(0, 4 * CHUNK, step=CHUNK)
    def _(r):
        base = sid * ROWS_PER_TILE + r
        pltpu.sync_copy(acc_sh.at[pl.ds(base, CHUNK)], stage_v)
        pltpu.sync_copy(stage_v, out_hbm.at[pl.ds(cid * N_PAD + base, CHUNK)])

    tbase = sid * ROWS_PER_TILE + 4 * CHUNK
    pltpu.sync_copy(acc_sh.at[pl.ds(tbase, TAIL)], stage_v.at[pl.ds(0, TAIL)])
    pltpu.sync_copy(stage_v.at[pl.ds(0, TAIL)],
                    out_hbm.at[pl.ds(cid * N_PAD + tbase, TAIL)])


def _scale1_body(x_ref, w_ref, d0_ref, d1_ref, xws_ref, dis_ref):
    deg = d0_ref[...] + d1_ref[...] + 1.0
    dis = lax.rsqrt(deg)
    xw = jnp.dot(x_ref[...], w_ref[...], precision=lax.Precision.HIGHEST,
                 preferred_element_type=jnp.float32)
    xws_ref[...] = xw * dis
    dis_ref[...] = dis


def _layer1(x_pad, W1, d0, d1):
    return pl.pallas_call(
        _scale1_body,
        grid=(N_PAD // BR,),
        in_specs=[
            pl.BlockSpec((BR, D_IN), lambda i: (i, 0)),
            pl.BlockSpec((D_IN, D_HID), lambda i: (0, 0)),
            pl.BlockSpec((BR, 1), lambda i: (i, 0)),
            pl.BlockSpec((BR, 1), lambda i: (i, 0)),
        ],
        out_specs=[
            pl.BlockSpec((BR, D_HID), lambda i: (i, 0)),
            pl.BlockSpec((BR, 1), lambda i: (i, 0)),
        ],
        out_shape=[
            jax.ShapeDtypeStruct((N_PAD, D_HID), jnp.float32),
            jax.ShapeDtypeStruct((N_PAD, 1), jnp.float32),
        ],
    )(x_pad, W1, d0, d1)


def _layer2_body(p0_ref, p1_ref, xws_ref, dis_ref, b1_ref, w2_ref, out_ref):
    s = p0_ref[...] + p1_ref[...] + xws_ref[...]
    pre = dis_ref[...] * s + b1_ref[...]
    h = jnp.maximum(pre, 0.0)
    xw2 = jnp.dot(h, w2_ref[...], precision=lax.Precision.HIGHEST,
                  preferred_element_type=jnp.float32)
    xw2s = xw2 * dis_ref[...]
    out_ref[...] = jnp.concatenate([xw2s, jnp.zeros_like(xw2s)], axis=1)


def _layer2(p0, p1, xw1s, dis2d, b1, W2):
    return pl.pallas_call(
        _layer2_body,
        grid=(N_PAD // BR,),
        in_specs=[
            pl.BlockSpec((BR, D_HID), lambda i: (i, 0)),
            pl.BlockSpec((BR, D_HID), lambda i: (i, 0)),
            pl.BlockSpec((BR, D_HID), lambda i: (i, 0)),
            pl.BlockSpec((BR, 1), lambda i: (i, 0)),
            pl.BlockSpec((1, D_HID), lambda i: (0, 0)),
            pl.BlockSpec((D_HID, N_CLASS), lambda i: (0, 0)),
        ],
        out_specs=pl.BlockSpec((BR, 2 * N_CLASS), lambda i: (i, 0)),
        out_shape=jax.ShapeDtypeStruct((N_PAD, 2 * N_CLASS), jnp.float32),
    )(p0, p1, xw1s, dis2d, b1, W2)


def _final_body(p0_ref, p1_ref, xws_ref, dis_ref, b2_ref, lsm_ref, sm_ref):
    s = (p0_ref[...] + p1_ref[...] + xws_ref[...])[:, :N_CLASS]
    logits = dis_ref[...] * s + b2_ref[...]
    m = jnp.max(logits, axis=1, keepdims=True)
    sh = logits - m
    ex = jnp.exp(sh)
    z = jnp.sum(ex, axis=1, keepdims=True)
    lsm_ref[...] = sh - jnp.log(z)
    sm_ref[...] = ex / z


def _final(p0, p1, xw2s, dis2d, b2):
    return pl.pallas_call(
        _final_body,
        grid=(N_PAD // BR,),
        in_specs=[
            pl.BlockSpec((BR, 2 * N_CLASS), lambda i: (i, 0)),
            pl.BlockSpec((BR, 2 * N_CLASS), lambda i: (i, 0)),
            pl.BlockSpec((BR, 2 * N_CLASS), lambda i: (i, 0)),
            pl.BlockSpec((BR, 1), lambda i: (i, 0)),
            pl.BlockSpec((1, N_CLASS), lambda i: (0, 0)),
        ],
        out_specs=[
            pl.BlockSpec((BR, N_CLASS), lambda i: (i, 0)),
            pl.BlockSpec((BR, N_CLASS), lambda i: (i, 0)),
        ],
        out_shape=[
            jax.ShapeDtypeStruct((N_PAD, N_CLASS), jnp.float32),
            jax.ShapeDtypeStruct((N_PAD, N_CLASS), jnp.float32),
        ],
    )(p0, p1, xw2s, dis2d, b2)


def kernel(features, edge_index, W1, b1, W2, b2):
    src = edge_index[0].astype(jnp.int32)
    dst = edge_index[1].astype(jnp.int32)
    pad = jnp.full((E_PAD - E,), PAD_NODE, jnp.int32)
    src_p = jnp.concatenate([src, pad]).reshape(NW, NCHUNKS, CHUNK)
    dst_p = jnp.concatenate([dst, pad]).reshape(NW, NCHUNKS, CHUNK)
    x_pad = jnp.pad(features, ((0, N_PAD - N), (0, 0)))

    zo = jnp.stack([jnp.zeros((CHUNK,), jnp.float32),
                    jnp.ones((CHUNK,), jnp.float32)])
    deg = _deg_kernel(dst_p, zo).reshape(NC, N_PAD)
    d0 = deg[0].reshape(N_PAD, 1)
    d1 = deg[1].reshape(N_PAD, 1)

    xw1s, dis2d = _layer1(x_pad, W1, d0, d1)

    z_hid = jnp.zeros((CHUNK, D_HID), jnp.float32)
    s1 = _agg_hid(xw1s, src_p, dst_p, z_hid)

    xw2s = _layer2(s1[0], s1[1], xw1s, dis2d, b1.reshape(1, D_HID), W2)

    s2 = _agg_hid(xw2s, src_p, dst_p, z_hid)

    lsm, sm = _final(s2[0], s2[1], xw2s, dis2d, b2.reshape(1, N_CLASS))
    return lsm[:N], sm[:N]
